# baseline (speedup 1.0000x reference)
import jax
import jax.numpy as jnp
from jax.experimental import pallas as pl
from jax.experimental.pallas import tpu as pltpu

_D_MODEL = 512
_D_STATE = 64
_HEADDIM = 64
_NHEADS = 16
_D_INNER = 1024
_CONV_DIM = _D_INNER + 2 * _D_STATE
_D_IN_PROJ = 2 * _D_INNER + 2 * _D_STATE + _NHEADS
_SEQLEN = 1024
_BATCH = 2
_CHUNK = 256
_NCHUNKS = _SEQLEN // _CHUNK
_LOG2E = 1.4426950408889634


def _layer_kernel(x_ref, w_in_ref, convw_ref, convb_ref, dtb_ref,
                  alog_ref, dbig_ref, normw_ref, w_out_ref, e64_ref, e256_ref,
                  out_ref, carry_ref, h_ref):
    c = pl.program_id(1)

    @pl.when(c == 0)
    def _():
        carry_ref[...] = jnp.zeros_like(carry_ref)
        h_ref[...] = jnp.zeros_like(h_ref)

    xb = x_ref[0]
    zxbcdt = jnp.dot(xb, w_in_ref[...],
                     preferred_element_type=jnp.float32)
    z = zxbcdt[:, :_D_INNER]
    xbc = zxbcdt[:, _D_INNER:_D_INNER + _CONV_DIM]
    dt_raw = zxbcdt[:, _D_INNER + _CONV_DIM:]
    dt = jax.nn.softplus(dt_raw + dtb_ref[...])

    xfull = jnp.concatenate([carry_ref[0:8], xbc], axis=0)
    carry_ref[0:8] = xbc[_CHUNK - 8:_CHUNK]
    s1 = xfull[7:7 + _CHUNK]
    P = xbc * convw_ref[3:4] + s1 * convw_ref[2:3]
    Q = xbc * convw_ref[1:2] + s1 * convw_ref[0:1]
    qfull = jnp.concatenate([carry_ref[8:16], Q], axis=0)
    carry_ref[8:16] = Q[_CHUNK - 8:_CHUNK]
    conv = P + qfull[6:6 + _CHUNK] + convb_ref[...]
    xconv = conv * jax.nn.sigmoid(conv)

    xs = xconv[:, :_D_INNER]
    Bmat = xconv[:, _D_INNER:_D_INNER + _D_STATE]
    Cmat = xconv[:, _D_INNER + _D_STATE:]

    A2 = -jnp.exp(alog_ref[...]) * _LOG2E
    a_log2 = dt * A2
    row = jax.lax.broadcasted_iota(jnp.int32, (_CHUNK, _CHUNK), 0)
    col = jax.lax.broadcasted_iota(jnp.int32, (_CHUNK, _CHUNK), 1)
    tril = (row >= col).astype(jnp.float32)
    cum2 = jnp.dot(tril, a_log2, preferred_element_type=jnp.float32)
    cum2_T = cum2.T
    cum2_last = cum2[_CHUNK - 1:_CHUNK, :]

    e64 = e64_ref[...]
    cumB = jnp.dot(cum2, e256_ref[...],
                   preferred_element_type=jnp.float32)
    stacked = jnp.concatenate(
        [dt, jnp.exp2(cum2), jnp.exp2(cum2_last - cum2)], axis=0)
    bcast = jnp.dot(stacked, e64, preferred_element_type=jnp.float32)
    dtB = bcast[:_CHUNK]
    ecumB = bcast[_CHUNK:2 * _CHUNK]
    decB = bcast[2 * _CHUNK:]
    hdec = ecumB[_CHUNK - 1:_CHUNK]

    Xdt = xs * dtB
    Xdec = Xdt * decB

    Yoff = jnp.dot(Cmat, h_ref[...],
                   preferred_element_type=jnp.float32) * ecumB
    h_ref[...] = h_ref[...] * hdec + jax.lax.dot_general(
        Bmat, Xdec, (((0,), (0,)), ((), ())),
        preferred_element_type=jnp.float32)

    CBm = jnp.dot(Cmat, Bmat.T, preferred_element_type=jnp.float32) * tril

    cs = _CHUNK // 2
    ys = []
    for hh in range(_NHEADS):
        base = _CHUNK * hh
        xcols = Xdt[:, _HEADDIM * hh:_HEADDIM * (hh + 1)]
        segA = cumB[:cs, base:base + cs] - cum2_T[hh:hh + 1, :cs]
        LA = jnp.exp2(jnp.minimum(segA, 0.0))
        ydA = jnp.dot(CBm[:cs, :cs] * LA, xcols[:cs],
                      preferred_element_type=jnp.float32)
        segB = cumB[cs:, base:base + _CHUNK] - cum2_T[hh:hh + 1, :]
        LB = jnp.exp2(jnp.minimum(segB, 0.0))
        ydB = jnp.dot(CBm[cs:, :] * LB, xcols,
                      preferred_element_type=jnp.float32)
        ys.append(jnp.concatenate([ydA, ydB], axis=0))
    Y = jnp.concatenate(ys, axis=1) + Yoff + xs * dbig_ref[...]

    g = Y * z * jax.nn.sigmoid(z)
    ms = jnp.mean(g * g, axis=1, keepdims=True)
    gn = g * jax.lax.rsqrt(ms + 1e-5) * normw_ref[...]
    out_ref[0] = jnp.dot(gn, w_out_ref[...],
                         preferred_element_type=jnp.float32)


def _run_layer(x, p, e64, e256):
    full = lambda shape: pl.BlockSpec(shape, lambda b, c: (0, 0))
    return pl.pallas_call(
        _layer_kernel,
        grid=(_BATCH, _NCHUNKS),
        in_specs=[
            pl.BlockSpec((1, _CHUNK, _D_MODEL), lambda b, c: (b, c, 0)),
            full((_D_MODEL, _D_IN_PROJ)),
            full((4, _CONV_DIM)),
            full((1, _CONV_DIM)),
            full((1, _NHEADS)),
            full((1, _NHEADS)),
            full((1, _D_INNER)),
            full((1, _D_INNER)),
            full((_D_INNER, _D_MODEL)),
            full((_NHEADS, _D_INNER)),
            full((_NHEADS, _NHEADS * _CHUNK)),
        ],
        out_specs=pl.BlockSpec((1, _CHUNK, _D_MODEL), lambda b, c: (b, c, 0)),
        out_shape=jax.ShapeDtypeStruct((_BATCH, _SEQLEN, _D_MODEL),
                                       jnp.float32),
        scratch_shapes=[
            pltpu.VMEM((16, _CONV_DIM), jnp.float32),
            pltpu.VMEM((_D_STATE, _NHEADS * _HEADDIM), jnp.float32),
        ],
        compiler_params=pltpu.CompilerParams(
            dimension_semantics=("arbitrary", "arbitrary"),
            vmem_limit_bytes=56 * 1024 * 1024,
        ),
    )(
        x,
        p["in_proj"],
        p["conv_w"].T,
        p["conv_b"].reshape(1, _CONV_DIM),
        p["dt_bias"].reshape(1, _NHEADS),
        p["A_log"].reshape(1, _NHEADS),
        jnp.repeat(p["D"], _HEADDIM).reshape(1, _D_INNER),
        p["norm_w"].reshape(1, _D_INNER),
        p["out_proj"],
        e64,
        e256,
    )


def _selector(block):
    lane = jnp.arange(_NHEADS * block, dtype=jnp.int32)[None, :]
    sub = jnp.arange(_NHEADS, dtype=jnp.int32)[:, None]
    return (lane // block == sub).astype(jnp.float32)


def kernel(x, params):
    e64 = _selector(_HEADDIM)
    e256 = _selector(_CHUNK)
    for p in params:
        x = _run_layer(x, p, e64, e256)
    return x


# speedup vs baseline: 1.0222x; 1.0222x over previous
import jax
import jax.numpy as jnp
from jax.experimental import pallas as pl
from jax.experimental.pallas import tpu as pltpu

_D_MODEL = 512
_D_STATE = 64
_HEADDIM = 64
_NHEADS = 16
_D_INNER = 1024
_CONV_DIM = _D_INNER + 2 * _D_STATE
_D_IN_PROJ = 2 * _D_INNER + 2 * _D_STATE + _NHEADS
_SEQLEN = 1024
_BATCH = 2
_CHUNK = 256
_NCHUNKS = _SEQLEN // _CHUNK
_LOG2E = 1.4426950408889634


def _layer_kernel(x_ref, w_in_ref, convw_ref, convb_ref, dtb_ref,
                  alog_ref, dbig_ref, normw_ref, w_out_ref, e64_ref, e128_ref,
                  out_ref, carry_ref, h_ref):
    c = pl.program_id(1)

    @pl.when(c == 0)
    def _():
        carry_ref[...] = jnp.zeros_like(carry_ref)
        h_ref[...] = jnp.zeros_like(h_ref)

    xb = x_ref[0]
    zxbcdt = jnp.dot(xb, w_in_ref[...],
                     preferred_element_type=jnp.float32)
    z = zxbcdt[:, :_D_INNER]
    xbc = zxbcdt[:, _D_INNER:_D_INNER + _CONV_DIM]
    dt_raw = zxbcdt[:, _D_INNER + _CONV_DIM:]
    dt = jax.nn.softplus(dt_raw + dtb_ref[...])

    xfull = jnp.concatenate([carry_ref[0:8], xbc], axis=0)
    carry_ref[0:8] = xbc[_CHUNK - 8:_CHUNK]
    s1 = xfull[7:7 + _CHUNK]
    P = xbc * convw_ref[3:4] + s1 * convw_ref[2:3]
    Q = xbc * convw_ref[1:2] + s1 * convw_ref[0:1]
    qfull = jnp.concatenate([carry_ref[8:16], Q], axis=0)
    carry_ref[8:16] = Q[_CHUNK - 8:_CHUNK]
    conv = P + qfull[6:6 + _CHUNK] + convb_ref[...]
    xconv = conv * jax.nn.sigmoid(conv)

    xs = xconv[:, :_D_INNER]
    Bmat = xconv[:, _D_INNER:_D_INNER + _D_STATE]
    Cmat = xconv[:, _D_INNER + _D_STATE:]

    sc = _CHUNK // 2
    A2 = -jnp.exp(alog_ref[...]) * _LOG2E
    a_log2 = dt * A2
    row = jax.lax.broadcasted_iota(jnp.int32, (_CHUNK, _CHUNK), 0)
    col = jax.lax.broadcasted_iota(jnp.int32, (_CHUNK, _CHUNK), 1)
    trilbd = ((row >= col) & (row // sc == col // sc)).astype(jnp.float32)
    cum2 = jnp.dot(trilbd, a_log2, preferred_element_type=jnp.float32)
    cum2_T = cum2.T
    last0 = cum2[sc - 1:sc, :]
    last1 = cum2[_CHUNK - 1:_CHUNK, :]

    e64 = e64_ref[...]
    cumB = jnp.dot(cum2, e128_ref[...],
                   preferred_element_type=jnp.float32)
    dec = jnp.exp2(jnp.concatenate(
        [last0 - cum2[:sc], last1 - cum2[sc:]], axis=0))
    stacked = jnp.concatenate([dt, jnp.exp2(cum2), dec], axis=0)
    bcast = jnp.dot(stacked, e64, preferred_element_type=jnp.float32)
    dtB = bcast[:_CHUNK]
    ecumB = bcast[_CHUNK:2 * _CHUNK]
    decB = bcast[2 * _CHUNK:]

    Xdt = xs * dtB
    Xdec = Xdt * decB

    tri = trilbd[:sc, :sc]
    cs = sc // 2
    h_cur = h_ref[...]
    ys = [[None, None] for _ in range(_NHEADS)]
    for sub in range(2):
        r0 = sub * sc
        sl = slice(r0, r0 + sc)
        Bs = Bmat[sl]
        Cs = Cmat[sl]
        Yoff_s = jnp.dot(Cs, h_cur,
                         preferred_element_type=jnp.float32) * ecumB[sl]
        h_cur = h_cur * ecumB[r0 + sc - 1:r0 + sc] + jax.lax.dot_general(
            Bs, Xdec[sl], (((0,), (0,)), ((), ())),
            preferred_element_type=jnp.float32)
        CBm = jnp.dot(Cs, Bs.T, preferred_element_type=jnp.float32) * tri
        for hh in range(_NHEADS):
            base = sc * hh
            xcols = Xdt[sl, _HEADDIM * hh:_HEADDIM * (hh + 1)]
            rowm = cum2_T[hh:hh + 1, sl]
            segA = cumB[r0:r0 + cs, base:base + cs] - rowm[:, :cs]
            LA = jnp.exp2(jnp.minimum(segA, 0.0))
            ydA = jnp.dot(CBm[:cs, :cs] * LA, xcols[:cs],
                          preferred_element_type=jnp.float32)
            segB = cumB[r0 + cs:r0 + sc, base:base + sc] - rowm
            LB = jnp.exp2(jnp.minimum(segB, 0.0))
            ydB = jnp.dot(CBm[cs:, :] * LB, xcols,
                          preferred_element_type=jnp.float32)
            yo = Yoff_s[:, _HEADDIM * hh:_HEADDIM * (hh + 1)]
            ys[hh][sub] = jnp.concatenate([ydA, ydB], axis=0) + yo
    h_ref[...] = h_cur
    Y = jnp.concatenate(
        [jnp.concatenate(ys[hh], axis=0) for hh in range(_NHEADS)],
        axis=1) + xs * dbig_ref[...]

    g = Y * z * jax.nn.sigmoid(z)
    ms = jnp.mean(g * g, axis=1, keepdims=True)
    gn = g * jax.lax.rsqrt(ms + 1e-5) * normw_ref[...]
    out_ref[0] = jnp.dot(gn, w_out_ref[...],
                         preferred_element_type=jnp.float32)


def _run_layer(x, p, e64, e128):
    full = lambda shape: pl.BlockSpec(shape, lambda b, c: (0, 0))
    return pl.pallas_call(
        _layer_kernel,
        grid=(_BATCH, _NCHUNKS),
        in_specs=[
            pl.BlockSpec((1, _CHUNK, _D_MODEL), lambda b, c: (b, c, 0)),
            full((_D_MODEL, _D_IN_PROJ)),
            full((4, _CONV_DIM)),
            full((1, _CONV_DIM)),
            full((1, _NHEADS)),
            full((1, _NHEADS)),
            full((1, _D_INNER)),
            full((1, _D_INNER)),
            full((_D_INNER, _D_MODEL)),
            full((_NHEADS, _D_INNER)),
            full((_NHEADS, _NHEADS * _CHUNK // 2)),
        ],
        out_specs=pl.BlockSpec((1, _CHUNK, _D_MODEL), lambda b, c: (b, c, 0)),
        out_shape=jax.ShapeDtypeStruct((_BATCH, _SEQLEN, _D_MODEL),
                                       jnp.float32),
        scratch_shapes=[
            pltpu.VMEM((16, _CONV_DIM), jnp.float32),
            pltpu.VMEM((_D_STATE, _NHEADS * _HEADDIM), jnp.float32),
        ],
        compiler_params=pltpu.CompilerParams(
            dimension_semantics=("arbitrary", "arbitrary"),
            vmem_limit_bytes=56 * 1024 * 1024,
        ),
    )(
        x,
        p["in_proj"],
        p["conv_w"].T,
        p["conv_b"].reshape(1, _CONV_DIM),
        p["dt_bias"].reshape(1, _NHEADS),
        p["A_log"].reshape(1, _NHEADS),
        jnp.repeat(p["D"], _HEADDIM).reshape(1, _D_INNER),
        p["norm_w"].reshape(1, _D_INNER),
        p["out_proj"],
        e64,
        e128,
    )


def _selector(block):
    lane = jnp.arange(_NHEADS * block, dtype=jnp.int32)[None, :]
    sub = jnp.arange(_NHEADS, dtype=jnp.int32)[:, None]
    return (lane // block == sub).astype(jnp.float32)


def kernel(x, params):
    e64 = _selector(_HEADDIM)
    e128 = _selector(_CHUNK // 2)
    for p in params:
        x = _run_layer(x, p, e64, e128)
    return x
